# baseline (speedup 1.0000x reference)
import functools

import jax
import jax.numpy as jnp
from jax import lax
from jax.experimental import pallas as pl
from jax.experimental.pallas import tpu as pltpu
from jax.experimental.pallas import tpu_sc as plsc

VOCAB = 1000
EMBED = 128
BATCH = 16384
SEQ = 20

NC = 2
NS = 16
LANES = 16
NW = NC * NS

NWORD = EMBED // 2
TPW = BATCH // NW
CH = 8
GR = CH * SEQ
NCHUNK = TPW // CH
NBUF = 4


def _fold_body(emb_ref, w_ref, b_ref, out_ref):
    tbl = emb_ref[...]
    rid = lax.broadcasted_iota(jnp.int32, tbl.shape, 0)
    tbl = jnp.where(rid == 0, 0.0, tbl)
    t2 = lax.dot_general(tbl, w_ref[...], (((1,), (1,)), ((), ())),
                         preferred_element_type=jnp.float32)
    t2 = (t2 + b_ref[...]) * (1.0 / SEQ)
    bits = pltpu.bitcast(t2, jnp.int32) + 0x8000
    packed = jnp.bitwise_or(
        lax.shift_right_logical(bits[:, :NWORD], 16),
        jnp.bitwise_and(bits[:, NWORD:], jnp.int32(-65536)))
    out_ref[...] = packed


_fold = pl.pallas_call(
    _fold_body,
    out_shape=jax.ShapeDtypeStruct((VOCAB, NWORD), jnp.int32),
)


def _gather_body(t2_hbm, tok_hbm, out_hbm, tok_v, buf_v, stage_v, sems):
    c = lax.axis_index("c")
    s = lax.axis_index("s")
    w = s * NC + c
    pltpu.sync_copy(tok_hbm.at[pl.ds(w * (TPW * SEQ), TPW * SEQ)], tok_v)

    def _gather_dma(i, buf):
        return pltpu.async_copy(
            t2_hbm.at[tok_v.at[pl.ds(i * GR, GR)]], buf_v.at[buf],
            sems.at[buf])

    def _drain(buf):
        pltpu.make_async_copy(t2_hbm.at[pl.ds(0, GR)], buf_v.at[buf],
                              sems.at[buf]).wait()

    for p in range(NBUF):
        _gather_dma(p, p)

    def th_body(th, carry):
        for ii in range(NBUF):
            i = th * NBUF + ii
            _drain(ii)

            @pl.when(th < (NCHUNK // NBUF) - 1)
            def _next(i=i, ii=ii):
                _gather_dma(i + NBUF, ii)

            def r_body(r, carry2, ii=ii):
                accs = [plsc.bitcast(
                            buf_v[ii, r * SEQ, pl.ds(k * LANES, LANES)],
                            jnp.bfloat16)
                        for k in range(4)]
                for l in range(1, SEQ):
                    for k in range(4):
                        accs[k] = accs[k] + plsc.bitcast(
                            buf_v[ii, r * SEQ + l, pl.ds(k * LANES, LANES)],
                            jnp.bfloat16)
                row = i * CH + r
                for k in range(4):
                    a = plsc.bitcast(accs[k], jnp.int32)
                    lo = plsc.bitcast(lax.shift_left(a, 16), jnp.float32)
                    hi = plsc.bitcast(
                        jnp.bitwise_and(a, jnp.int32(-65536)), jnp.float32)
                    stage_v[row, pl.ds(k * LANES, LANES)] = lo
                    stage_v[row, pl.ds(NWORD + k * LANES, LANES)] = hi
                return carry2

            lax.fori_loop(0, CH, r_body, 0)
        return carry

    lax.fori_loop(0, NCHUNK // NBUF, th_body, 0)
    pltpu.sync_copy(stage_v, out_hbm.at[pl.ds(w * TPW, TPW), :])


_gather = functools.partial(
    pl.kernel,
    out_type=jax.ShapeDtypeStruct((BATCH, EMBED), jnp.float32),
    mesh=plsc.VectorSubcoreMesh(core_axis_name="c", subcore_axis_name="s",
                                num_cores=NC, num_subcores=NS),
    scratch_types=[
        pltpu.VMEM((TPW * SEQ,), jnp.int32),
        pltpu.VMEM((NBUF, GR, NWORD), jnp.int32),
        pltpu.VMEM((TPW, EMBED), jnp.float32),
        pltpu.SemaphoreType.DMA((NBUF,)),
    ],
    compiler_params=pltpu.CompilerParams(needs_layout_passes=False, use_tc_tiling_on_sc=False),
)(_gather_body)


def kernel(tokens, emb_table, W, b):
    packed = _fold(emb_table, W, b.reshape(1, EMBED))
    return _gather(packed, tokens.astype(jnp.int32).reshape(-1))


# speedup vs baseline: 1.2594x; 1.2594x over previous
import functools

import jax
import jax.numpy as jnp
from jax import lax
from jax.experimental import pallas as pl
from jax.experimental.pallas import tpu as pltpu
from jax.experimental.pallas import tpu_sc as plsc

VOCAB = 1000
EMBED = 128
BATCH = 16384
SEQ = 20

NC = 2
NS = 16
LANES = 16
NW = NC * NS

NWORD = EMBED // 2
TPW = BATCH // NW
GROUPS = TPW // LANES
FUNROLL = 4


def _fold_body(emb_ref, w_ref, b_ref, out_ref):
    tbl = emb_ref[...]
    rid = lax.broadcasted_iota(jnp.int32, tbl.shape, 0)
    tbl = jnp.where(rid == 0, 0.0, tbl)
    t2 = lax.dot_general(tbl, w_ref[...], (((1,), (1,)), ((), ())),
                         preferred_element_type=jnp.float32)
    t2 = (t2 + b_ref[...]) * (1.0 / SEQ)
    bits = pltpu.bitcast(t2, jnp.int32) + 0x8000
    packed = jnp.bitwise_or(
        lax.shift_right_logical(bits[:, :NWORD], 16),
        jnp.bitwise_and(bits[:, NWORD:], jnp.int32(-65536)))
    out_ref[...] = packed


_fold = pl.pallas_call(
    _fold_body,
    out_shape=jax.ShapeDtypeStruct((VOCAB, NWORD), jnp.int32),
)


def _gather_body(t2_hbm, tok_hbm, out_hbm, table_v, tok_v,
                 stage0_v, stage1_v, osems):
    c = lax.axis_index("c")
    s = lax.axis_index("s")
    w = s * NC + c
    pltpu.sync_copy(t2_hbm, table_v)
    pltpu.sync_copy(tok_hbm.at[pl.ds(w * (TPW * SEQ), TPW * SEQ)], tok_v)

    piota = lax.iota(jnp.int32, LANES)
    stages = (stage0_v, stage1_v)

    def _out_dma(g, buf):
        return pltpu.make_async_copy(
            stages[buf],
            out_hbm.at[pl.ds(w * TPW + g * LANES, LANES), :],
            osems.at[buf])

    def th_body(th, carry):
        for ii in range(2):
            g = th * 2 + ii

            @pl.when(th >= 1)
            def _wait_prev(g=g, ii=ii):
                _out_dma(g - 2, ii).wait()

            tokrow = g * (LANES * SEQ) + piota * SEQ
            tb = [lax.shift_left(plsc.load_gather(tok_v, [tokrow + l]), 6)
                  for l in range(SEQ)]

            def f_body(f, carry2, tb=tb, ii=ii):
                for u in range(FUNROLL):
                    fw = f * FUNROLL + u
                    fvec = jnp.bitwise_and(fw + piota, NWORD - 1)
                    vals = [plsc.bitcast(
                                plsc.load_gather(table_v, [tb[l] + fvec]),
                                jnp.bfloat16)
                            for l in range(SEQ)]
                    while len(vals) > 1:
                        nxt = [vals[i] + vals[i + 1]
                               for i in range(0, len(vals) - 1, 2)]
                        if len(vals) % 2:
                            nxt.append(vals[-1])
                        vals = nxt
                    a = plsc.bitcast(vals[0], jnp.int32)
                    lo = plsc.bitcast(lax.shift_left(a, 16), jnp.float32)
                    hi = plsc.bitcast(
                        jnp.bitwise_and(a, jnp.int32(-65536)), jnp.float32)
                    plsc.store_scatter(stages[ii], [piota, fvec], lo)
                    plsc.store_scatter(stages[ii], [piota, fvec + NWORD],
                                       hi)
                return carry2

            lax.fori_loop(0, NWORD // FUNROLL, f_body, 0)
            _out_dma(g, ii).start()
        return carry

    lax.fori_loop(0, GROUPS // 2, th_body, 0)
    _out_dma(GROUPS - 2, 0).wait()
    _out_dma(GROUPS - 1, 1).wait()


_gather = functools.partial(
    pl.kernel,
    out_type=jax.ShapeDtypeStruct((BATCH, EMBED), jnp.float32),
    mesh=plsc.VectorSubcoreMesh(core_axis_name="c", subcore_axis_name="s",
                                num_cores=NC, num_subcores=NS),
    scratch_types=[
        pltpu.VMEM((VOCAB * NWORD,), jnp.int32),
        pltpu.VMEM((TPW * SEQ,), jnp.int32),
        pltpu.VMEM((LANES, EMBED), jnp.float32),
        pltpu.VMEM((LANES, EMBED), jnp.float32),
        pltpu.SemaphoreType.DMA((2,)),
    ],
    compiler_params=pltpu.CompilerParams(needs_layout_passes=False,
                                         use_tc_tiling_on_sc=False),
)(_gather_body)


def kernel(tokens, emb_table, W, b):
    packed = _fold(emb_table, W, b.reshape(1, EMBED))
    return _gather(packed.reshape(-1), tokens.astype(jnp.int32).reshape(-1))
